# baseline (speedup 1.0000x reference)
import functools

import jax
import jax.numpy as jnp
from jax import lax
from jax.experimental import pallas as pl
from jax.experimental.pallas import tpu as pltpu
from jax.experimental.pallas import tpu_sc as plsc

NC = 2
NS = 16
NW = NC * NS
NBUF = 5
LAG = 2


@functools.lru_cache(maxsize=None)
def _make_gather(batch: int, hist: int, d: int):
    assert batch % NW == 0
    bat_per_w = batch // NW
    assert bat_per_w <= 128
    assert hist % NBUF == 0

    mesh = plsc.VectorSubcoreMesh(core_axis_name="c", subcore_axis_name="s")

    @functools.partial(
        pl.kernel,
        mesh=mesh,
        out_type=jax.ShapeDtypeStruct((hist, batch, d), jnp.float32),
        scratch_types=[
            pltpu.VMEM((hist, bat_per_w), jnp.int32),
            pltpu.VMEM((NBUF, bat_per_w, d), jnp.float32),
        ]
        + [pltpu.SemaphoreType.DMA] * (2 * NBUF),
    )
    def gather(table_hbm, idxt_hbm, out_hbm, idx_v, rows_v, *sems):
        sg = sems[:NBUF]
        sw = sems[NBUF:]
        c = lax.axis_index("c")
        s = lax.axis_index("s")
        wid = s * NC + c
        bat0 = wid * bat_per_w
        depth = NBUF - LAG

        pltpu.sync_copy(idxt_hbm.at[:, wid], idx_v)

        for b in range(depth):
            pltpu.async_copy(table_hbm.at[idx_v.at[b]], rows_v.at[b], sg[b])

        def body(i, carry):
            g = i * NBUF
            for b in range(NBUF):
                j = g + b
                pltpu.make_async_copy(
                    table_hbm.at[idx_v.at[j]], rows_v.at[b], sg[b]
                ).wait()
                pltpu.async_copy(
                    rows_v.at[b], out_hbm.at[j, pl.ds(bat0, bat_per_w)], sw[b]
                )

                m = j + depth
                bm = (b + depth) % NBUF

                @pl.when(m < hist)
                def _():
                    @pl.when(m >= NBUF)
                    def _():
                        pltpu.make_async_copy(
                            rows_v.at[bm],
                            out_hbm.at[m - NBUF, pl.ds(bat0, bat_per_w)],
                            sw[bm],
                        ).wait()

                    pltpu.async_copy(
                        table_hbm.at[idx_v.at[m]], rows_v.at[bm], sg[bm]
                    )

            return carry

        lax.fori_loop(0, hist // NBUF, body, 0)

        for j in range(hist - NBUF, hist):
            b = j % NBUF
            pltpu.make_async_copy(
                rows_v.at[b], out_hbm.at[j, pl.ds(bat0, bat_per_w)], sw[b]
            ).wait()

    return gather


def kernel(indices, table):
    batch, hist = indices.shape
    _, d = table.shape
    idxt = jnp.transpose(indices).reshape(hist, NW, batch // NW)
    outt = _make_gather(batch, hist, d)(table, idxt)
    return jnp.transpose(outt, (1, 0, 2))


# speedup vs baseline: 1.0014x; 1.0014x over previous
import functools

import jax
import jax.numpy as jnp
from jax import lax
from jax.experimental import pallas as pl
from jax.experimental.pallas import tpu as pltpu
from jax.experimental.pallas import tpu_sc as plsc

NC = 2
NS = 16
NW = NC * NS
NBUF = 5
LAG = 1


@functools.lru_cache(maxsize=None)
def _make_gather(batch: int, hist: int, d: int):
    assert batch % NW == 0
    bat_per_w = batch // NW
    assert bat_per_w <= 128
    assert hist % NBUF == 0

    mesh = plsc.VectorSubcoreMesh(core_axis_name="c", subcore_axis_name="s")

    @functools.partial(
        pl.kernel,
        mesh=mesh,
        out_type=jax.ShapeDtypeStruct((hist, batch, d), jnp.float32),
        scratch_types=[
            pltpu.VMEM((hist, bat_per_w), jnp.int32),
            pltpu.VMEM((NBUF, bat_per_w, d), jnp.float32),
        ]
        + [pltpu.SemaphoreType.DMA] * (2 * NBUF),
    )
    def gather(table_hbm, idxt_hbm, out_hbm, idx_v, rows_v, *sems):
        sg = sems[:NBUF]
        sw = sems[NBUF:]
        c = lax.axis_index("c")
        s = lax.axis_index("s")
        wid = s * NC + c
        bat0 = wid * bat_per_w
        depth = NBUF - LAG

        pltpu.sync_copy(idxt_hbm.at[:, wid], idx_v)

        for b in range(depth):
            pltpu.async_copy(table_hbm.at[idx_v.at[b]], rows_v.at[b], sg[b])

        def body(i, carry):
            g = i * NBUF
            for b in range(NBUF):
                j = g + b
                pltpu.make_async_copy(
                    table_hbm.at[idx_v.at[j]], rows_v.at[b], sg[b]
                ).wait()
                pltpu.async_copy(
                    rows_v.at[b], out_hbm.at[j, pl.ds(bat0, bat_per_w)], sw[b]
                )

                m = j + depth
                bm = (b + depth) % NBUF

                @pl.when(m < hist)
                def _():
                    @pl.when(m >= NBUF)
                    def _():
                        pltpu.make_async_copy(
                            rows_v.at[bm],
                            out_hbm.at[m - NBUF, pl.ds(bat0, bat_per_w)],
                            sw[bm],
                        ).wait()

                    pltpu.async_copy(
                        table_hbm.at[idx_v.at[m]], rows_v.at[bm], sg[bm]
                    )

            return carry

        lax.fori_loop(0, hist // NBUF, body, 0)

        for j in range(hist - NBUF, hist):
            b = j % NBUF
            pltpu.make_async_copy(
                rows_v.at[b], out_hbm.at[j, pl.ds(bat0, bat_per_w)], sw[b]
            ).wait()

    return gather


def kernel(indices, table):
    batch, hist = indices.shape
    _, d = table.shape
    idxt = jnp.transpose(indices).reshape(hist, NW, batch // NW)
    outt = _make_gather(batch, hist, d)(table, idxt)
    return jnp.transpose(outt, (1, 0, 2))
